# baseline (speedup 1.0000x reference)
import functools

import jax
import jax.numpy as jnp
from jax import lax
from jax.experimental import pallas as pl
from jax.experimental.pallas import tpu as pltpu
from jax.experimental.pallas import tpu_sc as plsc

NC = 2
NS = 16
NW = NC * NS
W = 128
NBUF = 8
PF = 6
OBUF = 2
L = 16


@functools.lru_cache(maxsize=None)
def _build(BATCH: int, H: int, V: int, D: int):
    cols_w = BATCH // NW
    wpr = cols_w // W
    nwin = H * wpr
    nblk = nwin // NBUF
    assert nwin % NBUF == 0 and nblk >= 2 and D == 64

    mesh = plsc.VectorSubcoreMesh(core_axis_name="c", subcore_axis_name="s")

    @functools.partial(
        pl.kernel,
        mesh=mesh,
        out_type=jax.ShapeDtypeStruct((H, D // 8, BATCH // W, 8 * W),
                                      jnp.float32),
        compiler_params=pltpu.CompilerParams(use_tc_tiling_on_sc=False,
                                             needs_layout_passes=False),
        scratch_types=[
            pltpu.VMEM((H, cols_w), jnp.int32),
            pltpu.VMEM((NBUF, W), jnp.int32),
            pltpu.VMEM((NBUF, W, D), jnp.float32),
            pltpu.VMEM((OBUF, D // 8, 1, 8 * W), jnp.float32),
        ]
        + [pltpu.SemaphoreType.DMA] * (NBUF + OBUF + 1),
    )
    def gather_kernel(tp_hbm, xt_hbm, q_hbm, idxs, pidx, blk, outb, *sems):
        gsems, osems, isem = sems[:NBUF], sems[NBUF:NBUF + OBUF], sems[-1]
        wid = lax.axis_index("s") * NC + lax.axis_index("c")
        b0 = wid * cols_w
        pltpu.make_async_copy(
            xt_hbm.at[:, pl.ds(b0, cols_w)], idxs, isem).start()
        pltpu.make_async_copy(
            xt_hbm.at[:, pl.ds(b0, cols_w)], idxs, isem).wait()

        lanes = jnp.arange(L, dtype=jnp.int32)
        rows_list = [lanes + jnp.int32(k * L) for k in range(W // L)]

        def prep_fire(t, s):
            h, w = t // wpr, t % wpr
            for k in range(W // L):
                raw = idxs[h, pl.ds(w * W + k * L, L)]
                pidx[s, pl.ds(k * L, L)] = lax.shift_left(raw, 1)
            pltpu.make_async_copy(tp_hbm.at[pidx.at[s]], blk.at[s],
                                  gsems[s]).start()

        def drain_gather(s):
            pltpu.make_async_copy(tp_hbm.at[pidx.at[s]], blk.at[s],
                                  gsems[s]).wait()

        def out_copy(t, os):
            h, w = t // wpr, t % wpr
            return pltpu.make_async_copy(
                outb.at[os],
                q_hbm.at[h].at[:, pl.ds(wid * wpr + w, 1), :],
                osems[os])

        def transpose(t, s, os):
            zeros = lanes - lanes

            def sub(i, carry):
                k = lax.bitwise_and(i, jnp.int32(W // L - 1))
                c0 = lax.shift_left(
                    lax.shift_right_logical(i, jnp.int32(3)), 4)
                rows = lax.shift_left(k, 4) + lanes
                dvs, vals = [], []
                for j in range(L):
                    dv = c0 + lax.bitwise_and(lanes + jnp.int32(j),
                                              jnp.int32(L - 1))
                    dvs.append(dv)
                    vals.append(
                        plsc.load_gather(blk.at[s], [rows, dv]))
                for j in range(L):
                    dv = dvs[j]
                    av = lax.shift_right_logical(dv, 3)
                    slv = lax.shift_left(lax.bitwise_and(dv, jnp.int32(7)),
                                         7) + rows
                    plsc.store_scatter(outb.at[os], [av, zeros, slv],
                                       vals[j])
                return carry
            lax.fori_loop(0, (D // L) * (W // L), sub, jnp.int32(0))

        def iter_body(t, b, *, first_out, refire):
            drain_gather(b)
            os = b % OBUF
            if not first_out:
                out_copy(t - OBUF, os).wait()
            transpose(t, b, os)
            out_copy(t, os).start()
            if refire:
                prep_fire(t + PF, (b + PF) % NBUF)

        for b in range(PF):
            prep_fire(b, b)
        for b in range(NBUF):
            iter_body(b, b, first_out=(b < OBUF), refire=True)

        def blkbody(r, carry):
            for b in range(NBUF):
                iter_body(r * NBUF + b, b, first_out=False, refire=True)
            return carry

        lax.fori_loop(1, nblk - 1, blkbody, jnp.int32(0))

        for b in range(NBUF):
            t = (nblk - 1) * NBUF + b
            iter_body(t, b, first_out=False, refire=(b < NBUF - PF))
        for os in range(OBUF):
            out_copy((nblk - 1) * NBUF + (NBUF - OBUF) + os, os).wait()

    return gather_kernel


def kernel(x, embedding):
    bsz, hist = x.shape
    V, D = embedding.shape
    tab2 = jnp.pad(embedding, ((0, 0), (0, 128 - D))).reshape(2 * V, D)
    q4 = _build(bsz, hist, 2 * V, D)(tab2, x.T)
    return (q4.reshape(hist, D // 8, bsz // W, 8, W)
            .transpose(2, 4, 0, 1, 3)
            .reshape(bsz, hist, D))


# speedup vs baseline: 1.0064x; 1.0064x over previous
import functools

import jax
import jax.numpy as jnp
from jax import lax
from jax.experimental import pallas as pl
from jax.experimental.pallas import tpu as pltpu
from jax.experimental.pallas import tpu_sc as plsc

NC = 2
NS = 16
NW = NC * NS
W = 128
NBUF = 8
PF = 4
OBUF = 2
L = 16


@functools.lru_cache(maxsize=None)
def _build(BATCH: int, H: int, V: int, D: int):
    cols_w = BATCH // NW
    wpr = cols_w // W
    nwin = H * wpr
    nblk = nwin // NBUF
    assert nwin % NBUF == 0 and nblk >= 2 and D == 64

    mesh = plsc.VectorSubcoreMesh(core_axis_name="c", subcore_axis_name="s")

    @functools.partial(
        pl.kernel,
        mesh=mesh,
        out_type=jax.ShapeDtypeStruct((H, D // 8, BATCH // W, 8 * W),
                                      jnp.float32),
        compiler_params=pltpu.CompilerParams(use_tc_tiling_on_sc=False,
                                             needs_layout_passes=False),
        scratch_types=[
            pltpu.VMEM((H, cols_w), jnp.int32),
            pltpu.VMEM((NBUF, W), jnp.int32),
            pltpu.VMEM((NBUF, W, D), jnp.float32),
            pltpu.VMEM((OBUF, D // 8, 1, 8 * W), jnp.float32),
        ]
        + [pltpu.SemaphoreType.DMA] * (NBUF + OBUF + 1),
    )
    def gather_kernel(tp_hbm, xt_hbm, q_hbm, idxs, pidx, blk, outb, *sems):
        gsems, osems, isem = sems[:NBUF], sems[NBUF:NBUF + OBUF], sems[-1]
        wid = lax.axis_index("s") * NC + lax.axis_index("c")
        b0 = wid * cols_w
        pltpu.make_async_copy(
            xt_hbm.at[:, pl.ds(b0, cols_w)], idxs, isem).start()
        pltpu.make_async_copy(
            xt_hbm.at[:, pl.ds(b0, cols_w)], idxs, isem).wait()

        lanes = jnp.arange(L, dtype=jnp.int32)

        def prep_fire(t, s):
            h, w = t // wpr, t % wpr
            for k in range(W // L):
                raw = idxs[h, pl.ds(w * W + k * L, L)]
                pidx[s, pl.ds(k * L, L)] = lax.shift_left(raw, 1)
            pltpu.make_async_copy(tp_hbm.at[pidx.at[s]], blk.at[s],
                                  gsems[s]).start()

        def drain_gather(s):
            pltpu.make_async_copy(tp_hbm.at[pidx.at[s]], blk.at[s],
                                  gsems[s]).wait()

        def out_copy(t, os):
            h, w = t // wpr, t % wpr
            return pltpu.make_async_copy(
                outb.at[os],
                q_hbm.at[h].at[:, pl.ds(wid * wpr + w, 1), :],
                osems[os])

        def transpose(t, s, os):
            zeros = lanes - lanes

            def sub(i, carry):
                k = lax.bitwise_and(i, jnp.int32(W // L - 1))
                c0 = lax.shift_left(
                    lax.shift_right_logical(i, jnp.int32(3)), 4)
                rows = lax.shift_left(k, 4) + lanes
                dvs, vals = [], []
                for j in range(L):
                    dv = c0 + lax.bitwise_and(lanes + jnp.int32(j),
                                              jnp.int32(L - 1))
                    dvs.append(dv)
                    vals.append(
                        plsc.load_gather(blk.at[s], [rows, dv]))
                for j in range(L):
                    dv = dvs[j]
                    av = lax.shift_right_logical(dv, 3)
                    slv = lax.shift_left(lax.bitwise_and(dv, jnp.int32(7)),
                                         7) + rows
                    plsc.store_scatter(outb.at[os], [av, zeros, slv],
                                       vals[j])
                return carry
            lax.fori_loop(0, (D // L) * (W // L), sub, jnp.int32(0))

        def iter_body(t, b, *, first_out, refire):
            drain_gather(b)
            os = b % OBUF
            if not first_out:
                out_copy(t - OBUF, os).wait()
            transpose(t, b, os)
            out_copy(t, os).start()
            if refire:
                prep_fire(t + PF, (b + PF) % NBUF)

        for b in range(PF):
            prep_fire(b, b)
        for b in range(NBUF):
            iter_body(b, b, first_out=(b < OBUF), refire=True)

        def blkbody(r, carry):
            for b in range(NBUF):
                iter_body(r * NBUF + b, b, first_out=False, refire=True)
            return carry

        lax.fori_loop(1, nblk - 1, blkbody, jnp.int32(0))

        for b in range(NBUF):
            t = (nblk - 1) * NBUF + b
            iter_body(t, b, first_out=False, refire=(b < NBUF - PF))
        for os in range(OBUF):
            out_copy((nblk - 1) * NBUF + (NBUF - OBUF) + os, os).wait()

    return gather_kernel


def kernel(x, embedding):
    bsz, hist = x.shape
    V, D = embedding.shape
    tab2 = jnp.pad(embedding, ((0, 0), (0, 128 - D))).reshape(2 * V, D)
    q4 = _build(bsz, hist, 2 * V, D)(tab2, x.T)
    return (q4.reshape(hist, D // 8, bsz // W, 8, W)
            .transpose(2, 4, 0, 1, 3)
            .reshape(bsz, hist, D))
